# Initial kernel scaffold; baseline (speedup 1.0000x reference)
import jax
import jax.numpy as jnp
from jax.experimental import pallas as pl


def kernel(x, edge_index, pos_edge_index, neg_edge_index, W1, attn1, W2, attn2, Wp1, bp1, Wp2, bp2, Wp3, bp3):
    raise NotImplementedError("write your pallas kernel here")


# speedup vs baseline: 8.7505x; 8.7505x over previous
import functools

import jax
import jax.numpy as jnp
from jax import lax
from jax.experimental import pallas as pl
from jax.experimental.pallas import tpu as pltpu
from jax.experimental.pallas import tpu_sc as plsc

N = 10000
E = 160000
P = 10000
IN = 256
HID = 128
HEADS = 4
HH = HEADS * HID
TW = HH + 128

NC, NS, L = 2, 16, 16
NW = NC * NS
CHUNKR = 2500
CHUNK = 2560
NB = 4
NPAD = NB * CHUNK
SL = 160
FB = NPAD // SL
PASSES = 8
QB = FB // PASSES
M5 = 13108
EPT = E // NW
CAP = EPT
CAPP = CAP + 32
NBLK0 = (EPT + L - 1) // L
RPT = CHUNK // NS

ROW_BLK = 512
MUL = 26844


def _feat_body(x_ref, w_ref, a_ref, o_ref):
    f = jnp.dot(x_ref[...], w_ref[...], preferred_element_type=jnp.float32)
    o_ref[:, :HH] = f
    o_ref[:, HH:] = jnp.dot(f, a_ref[...], preferred_element_type=jnp.float32)


def _feat_table(x, w, a):
    k = x.shape[1]
    return pl.pallas_call(
        _feat_body,
        grid=(NPAD // ROW_BLK,),
        in_specs=[
            pl.BlockSpec((ROW_BLK, k), lambda i: (i, 0)),
            pl.BlockSpec((k, HH), lambda i: (0, 0)),
            pl.BlockSpec((HH, 128), lambda i: (0, 0)),
        ],
        out_specs=pl.BlockSpec((ROW_BLK, TW), lambda i: (i, 0)),
        out_shape=jax.ShapeDtypeStruct((NPAD, TW), jnp.float32),
    )(x, w, a)


def _elu(h):
    return jnp.where(h > 0, h, jnp.exp(jnp.minimum(h, 0.0)) - 1.0)


def _norm_feat_body(acc_ref, den_ref, ek_ref, res_ref, w_ref, a_ref,
                    h_ref, o_ref):
    rec = 1.0 / (jnp.abs(den_ref[...]) + 1e-9)
    recw = jnp.dot(rec, ek_ref[...], preferred_element_type=jnp.float32)
    h = acc_ref[...] * recw + res_ref[...]
    h = _elu(h)
    h_ref[...] = h
    f = jnp.dot(h, w_ref[...], preferred_element_type=jnp.float32)
    o_ref[:, :HH] = f
    o_ref[:, HH:] = jnp.dot(f, a_ref[...], preferred_element_type=jnp.float32)


def _norm_feat(acc, den, ek, res, w, a):
    return pl.pallas_call(
        _norm_feat_body,
        grid=(NPAD // ROW_BLK,),
        in_specs=[
            pl.BlockSpec((ROW_BLK, HH), lambda i: (i, 0)),
            pl.BlockSpec((ROW_BLK, L), lambda i: (i, 0)),
            pl.BlockSpec((L, HH), lambda i: (0, 0)),
            pl.BlockSpec((ROW_BLK, HH), lambda i: (i, 0)),
            pl.BlockSpec((HH, HH), lambda i: (0, 0)),
            pl.BlockSpec((HH, 128), lambda i: (0, 0)),
        ],
        out_specs=[
            pl.BlockSpec((ROW_BLK, HH), lambda i: (i, 0)),
            pl.BlockSpec((ROW_BLK, TW), lambda i: (i, 0)),
        ],
        out_shape=[
            jax.ShapeDtypeStruct((NPAD, HH), jnp.float32),
            jax.ShapeDtypeStruct((NPAD, TW), jnp.float32),
        ],
    )(acc, den, ek, res, w, a)


def _norm_body(acc_ref, den_ref, ek_ref, res_ref, h_ref):
    rec = 1.0 / (jnp.abs(den_ref[...]) + 1e-9)
    recw = jnp.dot(rec, ek_ref[...], preferred_element_type=jnp.float32)
    h_ref[...] = _elu(acc_ref[...] * recw + res_ref[...])


def _norm(acc, den, ek, res):
    return pl.pallas_call(
        _norm_body,
        grid=(NPAD // ROW_BLK,),
        in_specs=[
            pl.BlockSpec((ROW_BLK, HH), lambda i: (i, 0)),
            pl.BlockSpec((ROW_BLK, L), lambda i: (i, 0)),
            pl.BlockSpec((L, HH), lambda i: (0, 0)),
            pl.BlockSpec((ROW_BLK, HH), lambda i: (i, 0)),
        ],
        out_specs=pl.BlockSpec((ROW_BLK, HH), lambda i: (i, 0)),
        out_shape=jax.ShapeDtypeStruct((NPAD, HH), jnp.float32),
    )(acc, den, ek, res)


def _mlp_body(z_ref, w1_ref, b1_ref, w2_ref, b2_ref, w3_ref, b3_ref, o_ref):
    z = jnp.dot(z_ref[...], w1_ref[...], preferred_element_type=jnp.float32)
    z = jnp.maximum(z + b1_ref[...], 0.0)
    z = jnp.dot(z, w2_ref[...], preferred_element_type=jnp.float32)
    z = jnp.maximum(z + b2_ref[...], 0.0)
    o_ref[...] = jnp.dot(z, w3_ref[...], preferred_element_type=jnp.float32) \
        + b3_ref[...]


def _mlp(z, Wp1, bp1, Wp2, bp2, Wp3, bp3):
    n = z.shape[0]
    w3 = jnp.zeros((HID, 128), jnp.float32).at[:, 0].set(Wp3[:, 0])
    b3 = jnp.zeros((1, 128), jnp.float32).at[0, 0].set(bp3[0])
    out = pl.pallas_call(
        _mlp_body,
        grid=(n // ROW_BLK,),
        in_specs=[
            pl.BlockSpec((ROW_BLK, HH), lambda i: (i, 0)),
            pl.BlockSpec((HH, HID), lambda i: (0, 0)),
            pl.BlockSpec((1, HID), lambda i: (0, 0)),
            pl.BlockSpec((HID, HID), lambda i: (0, 0)),
            pl.BlockSpec((1, HID), lambda i: (0, 0)),
            pl.BlockSpec((HID, 128), lambda i: (0, 0)),
            pl.BlockSpec((1, 128), lambda i: (0, 0)),
        ],
        out_specs=pl.BlockSpec((ROW_BLK, 128), lambda i: (i, 0)),
        out_shape=jax.ShapeDtypeStruct((n, 128), jnp.float32),
    )(z, Wp1, bp1.reshape(1, HID), Wp2, bp2.reshape(1, HID), w3, b3)
    return out[:, :1]


_MESH = plsc.VectorSubcoreMesh(core_axis_name="c", subcore_axis_name="s")


def _bucket_body(src_h, dst_h, bsrc_h, bdst_h, cnt_h,
                 src_v, dst_v, bufs, cnt_v):
    c = lax.axis_index("c")
    s = lax.axis_index("s")
    w = s * NC + c
    pltpu.sync_copy(src_h.at[pl.ds(w * EPT, EPT)], src_v.at[pl.ds(0, EPT)])
    pltpu.sync_copy(dst_h.at[pl.ds(w * EPT, EPT)], dst_v.at[pl.ds(0, EPT)])

    zi = jnp.zeros((L,), jnp.int32)
    lanes = lax.iota(jnp.int32, L)
    last = jnp.full((L,), L - 1, jnp.int32)

    @pl.when(w == 0)
    def _tail():
        cnt_v[...] = zi
        pltpu.sync_copy(cnt_v, bsrc_h.at[pl.ds(FB * NW * CAP, L)])
        pltpu.sync_copy(cnt_v, bdst_h.at[pl.ds(FB * NW * CAP, L)])

    for p in range(PASSES):
        def zf(i, carry):
            for q in range(QB):
                bufs[2 * q][pl.ds(i * L, L)] = zi
                bufs[2 * q + 1][pl.ds(i * L, L)] = zi
            return carry

        lax.fori_loop(0, CAPP // L, zf, 0)

        def blk(i, cnts):
            sv = src_v[pl.ds(i * L, L)]
            dv = dst_v[pl.ds(i * L, L)]
            valid = (i * L + lanes) < EPT
            sb = jnp.right_shift(sv * MUL, 26)
            spad = sv + 60 * sb
            dp = dv + 60 * jnp.right_shift(dv * MUL, 26)
            t = jnp.right_shift(dp, 5)
            fb = jnp.right_shift(t * M5, 16)
            dl = dp - SL * fb
            qv = fb - QB * p
            out = []
            for q in range(QB):
                m = jnp.logical_and(qv == q, valid)
                cb = cnts[q]
                cs = plsc.cumsum(jnp.where(m, 1, 0))
                pos = jnp.where(m, cb - 1 + cs, CAP + lanes)
                plsc.store_scatter(bufs[2 * q], [pos], spad)
                plsc.store_scatter(bufs[2 * q + 1], [pos], dl)
                cnt_v[...] = cs
                out.append(cb + plsc.load_gather(cnt_v, [last]))
            return tuple(out)

        cnts = lax.fori_loop(0, NBLK0, blk,
                             tuple(zi for _ in range(QB)))

        cvec = zi
        for q in range(QB):
            cvec = jnp.where(lanes == q, cnts[q], cvec)
        cnt_v[...] = cvec
        pltpu.sync_copy(cnt_v, cnt_h.at[pl.ds(w * PASSES * L + p * L, L)])
        for q in range(QB):
            off = ((p * QB + q) * NW + w) * CAP
            pltpu.sync_copy(bufs[2 * q].at[pl.ds(0, CAP)],
                            bsrc_h.at[pl.ds(off, CAP)])
            pltpu.sync_copy(bufs[2 * q + 1].at[pl.ds(0, CAP)],
                            bdst_h.at[pl.ds(off, CAP)])


@functools.partial(
    pl.kernel,
    out_type=[
        jax.ShapeDtypeStruct((FB * NW * CAP + L,), jnp.int32),
        jax.ShapeDtypeStruct((FB * NW * CAP + L,), jnp.int32),
        jax.ShapeDtypeStruct((NW * PASSES * L,), jnp.int32),
    ],
    mesh=_MESH,
    compiler_params=pltpu.CompilerParams(needs_layout_passes=False),
    scratch_types=[
        pltpu.VMEM((NBLK0 * L,), jnp.int32),
        pltpu.VMEM((NBLK0 * L,), jnp.int32),
    ] + [pltpu.VMEM((CAPP,), jnp.int32) for _ in range(2 * QB)] + [
        pltpu.VMEM((L,), jnp.int32),
    ],
)
def _bucketize(src_h, dst_h, bsrc_h, bdst_h, cnt_h,
               src_v, dst_v, *rest):
    _bucket_body(src_h, dst_h, bsrc_h, bdst_h, cnt_h,
                 src_v, dst_v, list(rest[:-1]), rest[-1])


def _edge_body(tab_h, bsrc_h, bdst_h, cnt_h, attn_h, zrow_h, zden_h,
               acc_h, den_h,
               cnts_v, attn_v, tmp_v, fsrc, fdst,
               denrow, gsrc_i, gdst_i, dloc_i, acc, den, sem1, sem2):
    c = lax.axis_index("c")
    s = lax.axis_index("s")
    pltpu.sync_copy(cnt_h, cnts_v.at[pl.ds(0, NW * PASSES * L)])
    pltpu.sync_copy(attn_h, attn_v)
    lanes = lax.iota(jnp.int32, L)
    glanes = jnp.minimum(lanes, HEADS - 1) * L + (L - 1)

    for r in range(FB // (NC * NS)):
        fb = (NC * r + c) * NS + s
        pltpu.sync_copy(zrow_h, acc)
        pltpu.sync_copy(zden_h, den)
        pp = lax.div(fb, QB)
        qq = lax.rem(fb, QB)

        def wbody(w, carry):
            soff = (fb * NW + w) * CAP
            cnt = cnts_v[pl.ds(w * PASSES * L + pp * L + qq, L)][0]
            nblk = lax.div(cnt + (L - 1), L)

            def blk(i, carry2):
                base = i * L
                pltpu.sync_copy(bsrc_h.at[pl.ds(soff + base, L)], gsrc_i)
                pltpu.sync_copy(bdst_h.at[pl.ds(soff + base, L)], dloc_i)
                gdst_i[...] = dloc_i[...] + fb * SL
                cp1 = pltpu.async_copy(tab_h.at[gsrc_i], fsrc, sem1)
                cp2 = pltpu.async_copy(tab_h.at[gdst_i], fdst, sem2)
                cp1.wait()
                cp2.wait()
                cntv = jnp.full((L,), cnt, jnp.int32)

                def edge(e, carry3):
                    accs = [jnp.zeros((L,), jnp.float32)
                            for _ in range(HEADS)]
                    for j in range(HH // L):
                        fs = fsrc[e, pl.ds(j * L, L)]
                        fd = fdst[e, pl.ds(j * L, L)]
                        az = jnp.abs(fs + fd)
                        h = j // (HID // L)
                        accs[h] = accs[h] + az * attn_v[pl.ds(j * L, L)]
                    ssv = fsrc[e, pl.ds(HH, L)] + fdst[e, pl.ds(HH, L)]
                    for h in range(HEADS):
                        tmp_v[pl.ds(h * L, L)] = plsc.cumsum(accs[h])
                    rv = plsc.load_gather(tmp_v, [glanes])
                    logit = 0.6 * ssv + 0.4 * rv
                    ev = jnp.full((L,), base + e, jnp.int32)
                    mk = jnp.logical_and(lanes < HEADS, ev < cntv)
                    exv = jnp.where(mk, jnp.exp(logit), 0.0)
                    denrow[e, :] = exv
                    eidx = jnp.full((L,), e, jnp.int32)
                    dlv = plsc.load_gather(dloc_i, [eidx])
                    plsc.addupdate_scatter(den, [dlv, lanes], exv)
                    for h in range(HEADS):
                        exb = plsc.load_gather(
                            denrow, [eidx, jnp.full((L,), h, jnp.int32)])
                        for jj in range(HID // L):
                            cb = h * HID + jj * L
                            plsc.addupdate_scatter(
                                acc, [dlv, lanes + cb],
                                exb * fsrc[e, pl.ds(cb, L)])
                    return carry3

                lax.fori_loop(0, L, edge, 0)
                return carry2

            lax.fori_loop(0, nblk, blk, 0)
            return carry

        lax.fori_loop(0, NW, wbody, 0)
        pltpu.sync_copy(acc, acc_h.at[pl.ds(fb * SL, SL)])
        pltpu.sync_copy(den, den_h.at[pl.ds(fb * SL, SL)])


@functools.partial(
    pl.kernel,
    out_type=[
        jax.ShapeDtypeStruct((NPAD, HH), jnp.float32),
        jax.ShapeDtypeStruct((NPAD, L), jnp.float32),
    ],
    mesh=_MESH,
    compiler_params=pltpu.CompilerParams(needs_layout_passes=False),
    scratch_types=[
        pltpu.VMEM((NW * PASSES * L + L,), jnp.int32),
        pltpu.VMEM((HH,), jnp.float32),
        pltpu.VMEM((HEADS * L,), jnp.float32),
        pltpu.VMEM((L, TW), jnp.float32),
        pltpu.VMEM((L, TW), jnp.float32),
        pltpu.VMEM((L, L), jnp.float32),
        pltpu.VMEM((L,), jnp.int32),
        pltpu.VMEM((L,), jnp.int32),
        pltpu.VMEM((L,), jnp.int32),
        pltpu.VMEM((SL, HH), jnp.float32),
        pltpu.VMEM((SL, L), jnp.float32),
        pltpu.SemaphoreType.DMA,
        pltpu.SemaphoreType.DMA,
    ],
)
def _edge_pass(tab_h, bsrc_h, bdst_h, cnt_h, attn_h, zrow_h, zden_h,
               acc_h, den_h, *scratch):
    _edge_body(tab_h, bsrc_h, bdst_h, cnt_h, attn_h, zrow_h, zden_h,
               acc_h, den_h, *scratch)


PPT = 20480 // NW


def _pred_body(h_h, i0_h, i1_h, z_h, i0_v, i1_v, a_i, b_i, fa, fb, zbuf,
               sem1, sem2):
    c = lax.axis_index("c")
    s = lax.axis_index("s")
    w = s * NC + c
    pltpu.sync_copy(i0_h.at[pl.ds(w * PPT, PPT)], i0_v)
    pltpu.sync_copy(i1_h.at[pl.ds(w * PPT, PPT)], i1_v)

    def blk(i, carry):
        a_i[...] = i0_v[pl.ds(i * L, L)]
        b_i[...] = i1_v[pl.ds(i * L, L)]
        cp1 = pltpu.async_copy(h_h.at[a_i], fa, sem1)
        cp2 = pltpu.async_copy(h_h.at[b_i], fb, sem2)
        cp1.wait()
        cp2.wait()

        def pe(e, carry2):
            for j in range(HH // L):
                zbuf[e, pl.ds(j * L, L)] = \
                    fa[e, pl.ds(j * L, L)] * fb[e, pl.ds(j * L, L)]
            return carry2

        lax.fori_loop(0, L, pe, 0)
        pltpu.sync_copy(zbuf, z_h.at[pl.ds(w * PPT + i * L, L)])
        return carry

    lax.fori_loop(0, PPT // L, blk, 0)


@functools.partial(
    pl.kernel,
    out_type=jax.ShapeDtypeStruct((20480, HH), jnp.float32),
    mesh=_MESH,
    compiler_params=pltpu.CompilerParams(needs_layout_passes=False),
    scratch_types=[
        pltpu.VMEM((PPT,), jnp.int32),
        pltpu.VMEM((PPT,), jnp.int32),
        pltpu.VMEM((L,), jnp.int32),
        pltpu.VMEM((L,), jnp.int32),
        pltpu.VMEM((L, HH), jnp.float32),
        pltpu.VMEM((L, HH), jnp.float32),
        pltpu.VMEM((L, HH), jnp.float32),
        pltpu.SemaphoreType.DMA,
        pltpu.SemaphoreType.DMA,
    ],
)
def _pred_gather(h_h, i0_h, i1_h, z_h, *scratch):
    _pred_body(h_h, i0_h, i1_h, z_h, *scratch)


def _attn_mat(attn):
    rows = jnp.arange(HH)
    cols = rows // HID
    return jnp.zeros((HH, 128), jnp.float32).at[rows, cols].set(
        attn.reshape(-1))


def _ek_mat():
    cols = jnp.arange(HH)
    rows = cols // HID
    return jnp.zeros((L, HH), jnp.float32).at[rows, cols].set(1.0)


def _pad_nodes(x):
    k = x.shape[1]
    out = jnp.zeros((NB, CHUNK, k), x.dtype)
    out = out.at[:, :CHUNKR].set(x.reshape(NB, CHUNKR, k))
    return out.reshape(NPAD, k)


def _remap(idx):
    return idx + 60 * (idx // CHUNKR)


def kernel(x, edge_index, pos_edge_index, neg_edge_index, W1, attn1, W2,
           attn2, Wp1, bp1, Wp2, bp2, Wp3, bp3):
    src, dst = edge_index[0], edge_index[1]
    x_pad = _pad_nodes(x)
    ek = _ek_mat()
    zrow = jnp.zeros((SL, HH), jnp.float32)
    zden = jnp.zeros((SL, L), jnp.float32)
    zres = jnp.zeros((NPAD, HH), jnp.float32)

    bsrc, bdst, cnts = _bucketize(src, dst)

    tab1 = _feat_table(x_pad, W1, _attn_mat(attn1))
    acc1, den1 = _edge_pass(tab1, bsrc, bdst, cnts, attn1.reshape(-1),
                            zrow, zden)
    h1, tab2 = _norm_feat(acc1, den1, ek, zres, W2, _attn_mat(attn2))
    acc2, den2 = _edge_pass(tab2, bsrc, bdst, cnts, attn2.reshape(-1),
                            zrow, zden)
    h2 = _norm(acc2, den2, ek, h1)

    zpad = jnp.zeros((240,), jnp.int32)
    i0 = jnp.concatenate([_remap(pos_edge_index[0]), zpad,
                          _remap(neg_edge_index[0]), zpad])
    i1 = jnp.concatenate([_remap(pos_edge_index[1]), zpad,
                          _remap(neg_edge_index[1]), zpad])
    z = _pred_gather(h2, i0, i1)
    o = _mlp(z, Wp1, bp1, Wp2, bp2, Wp3, bp3)
    return (o[:P], o[10240:10240 + P])
